# baseline (speedup 1.0000x reference)
import functools

import jax
import jax.numpy as jnp
from jax import lax
from jax.experimental import pallas as pl
from jax.experimental.pallas import tpu as pltpu
from jax.experimental.pallas import tpu_sc as plsc

N = 10000
E = 320000
CIN = 128
F = 32
HALF = 16
EPS = 1e-5
NP = 16000
ER = E // 128
NC = 2
NS = 16
PSUB = NP // NS
PACK = N * F // 128
APAD = 2 * N + 32
F32 = jnp.float32

_MESH = plsc.VectorSubcoreMesh(core_axis_name="c", subcore_axis_name="s")


def _iota16():
    return lax.broadcasted_iota(jnp.int32, (16,), 0)


def _deg_body(src2, zeros1, ones_h, degb, idxb, ones_v, stg1, dvm, bb, dsem,
              acc1):
    cid = lax.axis_index("c")
    sid = lax.axis_index("s")
    pltpu.sync_copy(ones_h, ones_v)
    pltpu.sync_copy(zeros1, stg1)
    pltpu.sync_copy(stg1, acc1.at[pl.ds(sid * PSUB, PSUB)])
    plsc.subcore_barrier()

    base = cid * (ER // NC) + sid * 78
    pltpu.sync_copy(src2.at[pl.ds(base, 78)], idxb.at[pl.ds(0, 78)])

    @pl.when(sid < 2)
    def _stage_rem():
        row = cid * (ER // NC) + 16 * 78 + sid
        pltpu.sync_copy(src2.at[pl.ds(row, 1)], idxb.at[pl.ds(78, 1)])

    def _fire(d, c):
        pltpu.async_copy(ones_v, acc1.at[idxb.at[c]], dsem[d], add=True)

    def _drain(d):
        pltpu.make_async_copy(ones_v, acc1.at[idxb.at[0]], dsem[d]).wait()

    for d in range(6):
        _fire(d, d)

    @pl.loop(1, 13)
    def _round(r):
        for d in range(6):
            _drain(d)
            _fire(d, r * 6 + d)

    for d in range(6):
        _drain(d)

    @pl.when(sid < 2)
    def _rem():
        pltpu.async_copy(ones_v, acc1.at[idxb.at[78]], dsem[0], add=True)
        pltpu.make_async_copy(ones_v, acc1.at[idxb.at[78]], dsem[0]).wait()

    plsc.subcore_barrier()

    @pl.when(sid < 10)
    def _emit():
        pltpu.sync_copy(acc1.at[pl.ds(sid * PSUB, PSUB)], dvm)

        @pl.loop(0, 250)
        def _row(r):
            for l in range(8):
                node = r * 4 + l // 2
                vals = plsc.load_gather(dvm, [jnp.full((16,), node, jnp.int32)])
                bb[r, pl.ds(l * 16, 16)] = vals

        pltpu.sync_copy(bb, degb.at[pl.ds(cid * 2500 + sid * 250, 250)])


@functools.partial(
    pl.kernel,
    out_type=jax.ShapeDtypeStruct((NC * 2500, 128), F32),
    mesh=_MESH,
    scratch_types=[
        pltpu.VMEM((80, 128), jnp.int32),
        pltpu.VMEM((128,), F32),
        pltpu.VMEM((PSUB,), F32),
        pltpu.VMEM((PSUB,), F32),
        pltpu.VMEM((250, 128), F32),
        [pltpu.SemaphoreType.DMA] * 6,
        pltpu.VMEM_SHARED((NP,), F32),
    ],
    compiler_params=pltpu.CompilerParams(use_tc_tiling_on_sc=False,
                                        needs_layout_passes=False),
)
def _deg_kernel(src2, zeros1, ones_h, degb, idxb, ones_v, stg1, dvm, bb, dsem,
                acc1):
    _deg_body(src2, zeros1, ones_h, degb, idxb, ones_v, stg1, dvm, bb, dsem,
              acc1)


_D = 6
_NCH = 39


def _gs_body(table, src2, dst2, zeros2, agg, srca, dsta, stg2, oidx, rows,
             gsem, ssem, acc):
    cid = lax.axis_index("c")
    sid = lax.axis_index("s")
    pltpu.sync_copy(zeros2, stg2)
    pltpu.sync_copy(stg2.at[pl.ds(0, PSUB)], acc.at[pl.ds(sid * PSUB, PSUB)])

    base = sid * _NCH
    pltpu.sync_copy(src2.at[pl.ds(base, _NCH)], srca.at[pl.ds(0, _NCH)])
    pltpu.sync_copy(dst2.at[pl.ds(base, _NCH)], dsta.at[pl.ds(0, _NCH)])

    @pl.when(sid == 0)
    def _stage_rem():
        pltpu.sync_copy(src2.at[pl.ds(16 * _NCH, 1)], srca.at[pl.ds(_NCH, 1)])
        pltpu.sync_copy(dst2.at[pl.ds(16 * _NCH, 1)], dsta.at[pl.ds(_NCH, 1)])

    @pl.loop(0, _NCH + 1)
    def _adj(r):
        for l in range(32):
            v = srca[r, pl.ds(l * 16, 16)]
            srca[r, pl.ds(l * 16, 16)] = v * 2 + cid

    plsc.subcore_barrier()

    def _start_gather(d, chunk):
        pltpu.async_copy(table.at[srca.at[chunk]], rows[d], gsem[d])

    def _wait_gather(d):
        pltpu.make_async_copy(table.at[srca.at[0]], rows[d], gsem[d]).wait()

    def _start_scatter(d, chunk):
        pltpu.async_copy(rows[d], acc.at[dsta.at[chunk]], ssem[d], add=True)

    def _wait_scatter(d):
        pltpu.make_async_copy(rows[d], acc.at[dsta.at[0]], ssem[d]).wait()

    for d in range(_D):
        _start_gather(d, d)

    @pl.loop(0, _NCH // _D)
    def _round(r):
        for d in range(_D):
            chunk = r * _D + d
            _wait_gather(d)
            _start_scatter(d, chunk)
            _wait_scatter(d)

            @pl.when(chunk + _D < _NCH)
            def _next():
                _start_gather(d, chunk + _D)

    for d in range(_NCH - _D * (_NCH // _D)):
        chunk = _D * (_NCH // _D) + d
        _wait_gather(d)
        _start_scatter(d, chunk)
        _wait_scatter(d)

    @pl.when(sid == 0)
    def _rem():
        _start_gather(0, _NCH)
        _wait_gather(0)
        _start_scatter(0, _NCH)
        _wait_scatter(0)

    plsc.subcore_barrier()

    @pl.when(sid < 10)
    def _emit():
        pltpu.sync_copy(acc.at[pl.ds(sid * PSUB, PSUB)],
                        stg2.at[pl.ds(0, PSUB)])
        nodebase = sid * PSUB
        dump = 2 * N + cid * NS + sid
        for c in range(8):
            for l in range(8):
                pos = c * 128 + l * 16 + _iota16()
                idx = jnp.where(pos < PSUB, 2 * (nodebase + pos) + cid, dump)
                oidx[c, pl.ds(l * 16, 16)] = idx
        sems = (gsem + ssem)[:8]
        for c in range(8):
            pltpu.async_copy(stg2.at[pl.ds(c * 128, 128)],
                             agg.at[oidx.at[c]], sems[c])
        for c in range(8):
            pltpu.make_async_copy(stg2.at[pl.ds(c * 128, 128)],
                                  agg.at[oidx.at[0]], sems[c]).wait()


@functools.partial(
    pl.kernel,
    out_type=jax.ShapeDtypeStruct((APAD, HALF), F32),
    mesh=_MESH,
    scratch_types=[
        pltpu.VMEM((_NCH + 1, 512), jnp.int32),
        pltpu.VMEM((_NCH + 1, 512), jnp.int32),
        pltpu.VMEM((1024, HALF), F32),
        pltpu.VMEM((8, 128), jnp.int32),
        [pltpu.VMEM((512, HALF), F32)] * _D,
        [pltpu.SemaphoreType.DMA] * _D,
        [pltpu.SemaphoreType.DMA] * _D,
        pltpu.VMEM_SHARED((NP, HALF), F32),
    ],
    compiler_params=pltpu.CompilerParams(use_tc_tiling_on_sc=False,
                                        needs_layout_passes=False),
)
def _gs_kernel(table, src2, dst2, zeros2, agg, srca, dsta, stg2, oidx, rows,
               gsem, ssem, acc):
    _gs_body(table, src2, dst2, zeros2, agg, srca, dsta, stg2, oidx, rows,
             gsem, ssem, acc)


PK = 2504
PV = 2500


def _dinv_full(degb):
    deg = degb[0:PV] + degb[PV:2 * PV]
    dinv = jnp.where(deg > 0, lax.rsqrt(deg), 0.0)
    return jnp.concatenate([dinv, jnp.zeros((PK - PV, 128), F32)], axis=0)


def _pick(y, off):
    return jnp.concatenate(
        [y[:, 96 * k + off:96 * k + off + F] for k in range(4)], axis=1)


def _row_pad(a):
    return jnp.concatenate([a, jnp.zeros((PK - PV, 128), F32)], axis=0)


def _mm1_body(xp_ref, w_ref, db_ref, z_ref, t_ref, g_ref, dv_ref):
    xp = xp_ref[...]
    w = w_ref[...]
    ys = [jnp.dot(xp[:, 128 * k:128 * (k + 1)], w,
                  preferred_element_type=F32,
                  precision=lax.Precision.HIGHEST) for k in range(4)]
    dinv = _dinv_full(db_ref[...])
    dv_ref[...] = dinv

    def pick(off):
        return jnp.concatenate([y[:, off:off + F] for y in ys], axis=1)

    z_ref[...] = _row_pad(pick(0))
    t_ref[...] = _row_pad(pick(F))
    g_ref[...] = _row_pad(dinv[0:PV] * pick(2 * F))


def _mid_body(t_ref, a_ref, dv_ref, o_ref):
    dinv = dv_ref[...]
    o_ref[...] = dinv * t_ref[...] - 2.0 * dinv * dinv * a_ref[...]


def _row_mask():
    return lax.broadcasted_iota(jnp.int32, (PK, 128), 0) < PV


def _post_body(z_ref, a_ref, dv_ref, b_ref, r_ref, p_ref, q_ref):
    y = z_ref[...] - dv_ref[...] * a_ref[...] + b_ref[...]
    r = jnp.maximum(y, 0.0)
    r_ref[...] = r
    rm = jnp.where(_row_mask(), r, 0.0)
    p_ref[...] = jnp.broadcast_to(jnp.sum(rm, axis=0, keepdims=True),
                                  (8, 128))
    q_ref[...] = jnp.broadcast_to(jnp.sum(rm * rm, axis=0, keepdims=True),
                                  (8, 128))


def _fold4(s):
    return (s[:, 0:32] + s[:, 32:64] + s[:, 64:96] + s[:, 96:128])


def _rep4(s):
    return jnp.concatenate([s, s, s, s], axis=1)


def _stats(p_ref, q_ref):
    mu = _rep4(_fold4(jnp.sum(p_ref[...], axis=0, keepdims=True))) * (
        1.0 / (8 * N))
    ms = _rep4(_fold4(jnp.sum(q_ref[...], axis=0, keepdims=True))) * (
        1.0 / (8 * N))
    var = ms - mu * mu
    return mu, lax.rsqrt(var + EPS)


def _mm2_body(r_ref, p_ref, q_ref, w_ref, dv_ref, z_ref, t_ref, g_ref):
    mu, sinv = _stats(p_ref, q_ref)
    h = (r_ref[...] - mu) * sinv
    w = w_ref[...]
    ys = [jnp.dot(h[:, F * k:F * (k + 1)], w, preferred_element_type=F32,
                  precision=lax.Precision.HIGHEST) for k in range(4)]

    def pick(off):
        return jnp.concatenate([y[:, off:off + F] for y in ys], axis=1)

    z_ref[...] = pick(0)
    t_ref[...] = pick(F)
    g_ref[...] = dv_ref[...] * pick(2 * F)


def _norm_body(r_ref, p_ref, q_ref, o_ref):
    mu, sinv = _stats(p_ref, q_ref)
    o_ref[...] = (r_ref[...] - mu) * sinv


def _full(shape):
    return pl.BlockSpec(shape, lambda: tuple(0 for _ in shape))


def _pk_out(n_arrs):
    return [jax.ShapeDtypeStruct((PK, 128), F32) for _ in range(n_arrs)]


_PKS = _full((PK, 128))
_DBS = _full((2 * PV, 128))
_PQS = _full((8, 128))

_mm1_call = pl.pallas_call(
    _mm1_body,
    in_specs=[_full((PV, 4 * CIN)), _full((CIN, 3 * F)), _DBS],
    out_specs=[_PKS] * 4,
    out_shape=_pk_out(4),
)

_mid_call = pl.pallas_call(
    _mid_body,
    in_specs=[_PKS, _PKS, _PKS],
    out_specs=[_PKS],
    out_shape=_pk_out(1),
)

_post_call = pl.pallas_call(
    _post_body,
    in_specs=[_PKS, _PKS, _PKS, _full((1, 128))],
    out_specs=[_PKS, _PQS, _PQS],
    out_shape=_pk_out(1) + [jax.ShapeDtypeStruct((8, 128), F32)] * 2,
)

_mm2_call = pl.pallas_call(
    _mm2_body,
    in_specs=[_PKS, _PQS, _PQS, _full((F, 3 * F)), _PKS],
    out_specs=[_PKS] * 3,
    out_shape=_pk_out(3),
)

_norm_call = pl.pallas_call(
    _norm_body,
    in_specs=[_PKS, _PQS, _PQS],
    out_specs=[_PKS],
    out_shape=_pk_out(1),
)


def kernel(x, edge_index, W1_0, W1_1, W1_2, b1, W2_0, W2_1, W2_2, b2):
    src2 = edge_index[0].reshape(ER, 128)
    dst2 = edge_index[1].reshape(ER, 128)
    zeros1 = jnp.zeros((PSUB,), F32)
    ones_h = jnp.ones((128,), F32)
    zeros2 = jnp.zeros((1024, HALF), F32)

    degb = _deg_kernel(src2, zeros1, ones_h)

    src5 = edge_index[0].reshape(625, 512)
    dst5 = edge_index[1].reshape(625, 512)

    def gs(gin):
        agg = _gs_kernel(gin.reshape(APAD, HALF), src5, dst5, zeros2)
        return agg.reshape(PK, 128)

    wcat1 = jnp.concatenate([W1_0 - W1_2, W1_1, W1_2], axis=1)
    xp = x.reshape(PV, 4 * CIN)
    b1p = jnp.tile(b1, 4).reshape(1, 128)
    b2p = jnp.tile(b2, 4).reshape(1, 128)
    wcat2 = jnp.concatenate([W2_0 - W2_2, W2_1, W2_2], axis=1)

    z0, t1, gin1, dinvp = _mm1_call(xp, wcat1, degb)
    agg1 = gs(gin1)
    (gin2,) = _mid_call(t1, agg1, dinvp)
    agg2 = gs(gin2)
    r1, p1, q1 = _post_call(z0, agg2, dinvp, b1p)

    z0b, t2, gin3 = _mm2_call(r1, p1, q1, wcat2, dinvp)
    agg3 = gs(gin3)
    (gin4,) = _mid_call(t2, agg3, dinvp)
    agg4 = gs(gin4)
    r2, p2, q2 = _post_call(z0b, agg4, dinvp, b2p)

    (outp,) = _norm_call(r2, p2, q2)
    return outp[:PV].reshape(N, F)


# speedup vs baseline: 1.0146x; 1.0146x over previous
import functools

import jax
import jax.numpy as jnp
from jax import lax
from jax.experimental import pallas as pl
from jax.experimental.pallas import tpu as pltpu
from jax.experimental.pallas import tpu_sc as plsc

N = 10000
E = 320000
CIN = 128
F = 32
HALF = 16
EPS = 1e-5
NP = 16000
ER = E // 128
NC = 2
NS = 16
PSUB = NP // NS
PACK = N * F // 128
APAD = 2 * N + 32
F32 = jnp.float32

_MESH = plsc.VectorSubcoreMesh(core_axis_name="c", subcore_axis_name="s")


def _iota16():
    return lax.broadcasted_iota(jnp.int32, (16,), 0)


def _deg_body(src2, zeros1, ones_h, degb, idxb, ones_v, stg1, dvm, bb, dsem,
              acc1):
    cid = lax.axis_index("c")
    sid = lax.axis_index("s")
    pltpu.sync_copy(ones_h, ones_v)
    pltpu.sync_copy(zeros1, stg1)
    pltpu.sync_copy(stg1, acc1.at[pl.ds(sid * PSUB, PSUB)])
    plsc.subcore_barrier()

    base = cid * (ER // NC) + sid * 78
    pltpu.sync_copy(src2.at[pl.ds(base, 78)], idxb.at[pl.ds(0, 78)])

    @pl.when(sid < 2)
    def _stage_rem():
        row = cid * (ER // NC) + 16 * 78 + sid
        pltpu.sync_copy(src2.at[pl.ds(row, 1)], idxb.at[pl.ds(78, 1)])

    def _fire(d, c):
        pltpu.async_copy(ones_v, acc1.at[idxb.at[c]], dsem[d], add=True)

    def _drain(d):
        pltpu.make_async_copy(ones_v, acc1.at[idxb.at[0]], dsem[d]).wait()

    for d in range(6):
        _fire(d, d)

    @pl.loop(1, 13)
    def _round(r):
        for d in range(6):
            _drain(d)
            _fire(d, r * 6 + d)

    for d in range(6):
        _drain(d)

    @pl.when(sid < 2)
    def _rem():
        pltpu.async_copy(ones_v, acc1.at[idxb.at[78]], dsem[0], add=True)
        pltpu.make_async_copy(ones_v, acc1.at[idxb.at[78]], dsem[0]).wait()

    plsc.subcore_barrier()

    @pl.when(sid < 10)
    def _emit():
        pltpu.sync_copy(acc1.at[pl.ds(sid * PSUB, PSUB)], dvm)

        @pl.loop(0, 250)
        def _row(r):
            for l in range(8):
                node = r * 4 + l // 2
                vals = plsc.load_gather(dvm, [jnp.full((16,), node, jnp.int32)])
                bb[r, pl.ds(l * 16, 16)] = vals

        pltpu.sync_copy(bb, degb.at[pl.ds(cid * 2500 + sid * 250, 250)])


@functools.partial(
    pl.kernel,
    out_type=jax.ShapeDtypeStruct((NC * 2500, 128), F32),
    mesh=_MESH,
    scratch_types=[
        pltpu.VMEM((80, 128), jnp.int32),
        pltpu.VMEM((128,), F32),
        pltpu.VMEM((PSUB,), F32),
        pltpu.VMEM((PSUB,), F32),
        pltpu.VMEM((250, 128), F32),
        [pltpu.SemaphoreType.DMA] * 6,
        pltpu.VMEM_SHARED((NP,), F32),
    ],
    compiler_params=pltpu.CompilerParams(use_tc_tiling_on_sc=False,
                                        needs_layout_passes=False),
)
def _deg_kernel(src2, zeros1, ones_h, degb, idxb, ones_v, stg1, dvm, bb, dsem,
                acc1):
    _deg_body(src2, zeros1, ones_h, degb, idxb, ones_v, stg1, dvm, bb, dsem,
              acc1)


_D = 6
_NCH = 39


def _gs_body(table, src2, dst2, zeros2, agg, srca, dsta, stg2, oidx, rows,
             gsem, ssem, acc):
    cid = lax.axis_index("c")
    sid = lax.axis_index("s")
    pltpu.sync_copy(zeros2, stg2)
    pltpu.sync_copy(stg2.at[pl.ds(0, PSUB)], acc.at[pl.ds(sid * PSUB, PSUB)])

    base = sid * _NCH
    pltpu.sync_copy(src2.at[pl.ds(base, _NCH)], srca.at[pl.ds(0, _NCH)])
    pltpu.sync_copy(dst2.at[pl.ds(base, _NCH)], dsta.at[pl.ds(0, _NCH)])

    @pl.when(sid == 0)
    def _stage_rem():
        pltpu.sync_copy(src2.at[pl.ds(16 * _NCH, 1)], srca.at[pl.ds(_NCH, 1)])
        pltpu.sync_copy(dst2.at[pl.ds(16 * _NCH, 1)], dsta.at[pl.ds(_NCH, 1)])

    @pl.loop(0, _NCH + 1)
    def _adj(r):
        for l in range(32):
            v = srca[r, pl.ds(l * 16, 16)]
            srca[r, pl.ds(l * 16, 16)] = v * 2 + cid

    plsc.subcore_barrier()

    def _start_gather(d, chunk):
        pltpu.async_copy(table.at[srca.at[chunk]], rows[d], gsem[d])

    def _wait_gather(d):
        pltpu.make_async_copy(table.at[srca.at[0]], rows[d], gsem[d]).wait()

    def _start_scatter(d, chunk):
        pltpu.async_copy(rows[d], acc.at[dsta.at[chunk]], ssem[d], add=True)

    def _wait_scatter(d):
        pltpu.make_async_copy(rows[d], acc.at[dsta.at[0]], ssem[d]).wait()

    for d in range(_D):
        _start_gather(d, d)

    @pl.loop(0, _NCH // _D)
    def _round(r):
        for d in range(_D):
            chunk = r * _D + d
            _wait_gather(d)
            _start_scatter(d, chunk)
            _wait_scatter(d)

            @pl.when(chunk + _D < _NCH)
            def _next():
                _start_gather(d, chunk + _D)

    for d in range(_NCH - _D * (_NCH // _D)):
        chunk = _D * (_NCH // _D) + d
        _wait_gather(d)
        _start_scatter(d, chunk)
        _wait_scatter(d)

    @pl.when(sid == 0)
    def _rem():
        _start_gather(0, _NCH)
        _wait_gather(0)
        _start_scatter(0, _NCH)
        _wait_scatter(0)

    plsc.subcore_barrier()

    @pl.when(sid < 10)
    def _emit():
        pltpu.sync_copy(acc.at[pl.ds(sid * PSUB, PSUB)],
                        stg2.at[pl.ds(0, PSUB)])
        nodebase = sid * PSUB
        dump = 2 * N + cid * NS + sid
        for c in range(8):
            for l in range(8):
                pos = c * 128 + l * 16 + _iota16()
                idx = jnp.where(pos < PSUB, 2 * (nodebase + pos) + cid, dump)
                oidx[c, pl.ds(l * 16, 16)] = idx
        sems = (gsem + ssem)[:8]
        for c in range(8):
            pltpu.async_copy(stg2.at[pl.ds(c * 128, 128)],
                             agg.at[oidx.at[c]], sems[c])
        for c in range(8):
            pltpu.make_async_copy(stg2.at[pl.ds(c * 128, 128)],
                                  agg.at[oidx.at[0]], sems[c]).wait()


@functools.partial(
    pl.kernel,
    out_type=jax.ShapeDtypeStruct((APAD, HALF), F32),
    mesh=_MESH,
    scratch_types=[
        pltpu.VMEM((_NCH + 1, 512), jnp.int32),
        pltpu.VMEM((_NCH + 1, 512), jnp.int32),
        pltpu.VMEM((1024, HALF), F32),
        pltpu.VMEM((8, 128), jnp.int32),
        [pltpu.VMEM((512, HALF), F32)] * _D,
        [pltpu.SemaphoreType.DMA] * _D,
        [pltpu.SemaphoreType.DMA] * _D,
        pltpu.VMEM_SHARED((NP, HALF), F32),
    ],
    compiler_params=pltpu.CompilerParams(use_tc_tiling_on_sc=False,
                                        needs_layout_passes=False),
)
def _gs_kernel(table, src2, dst2, zeros2, agg, srca, dsta, stg2, oidx, rows,
               gsem, ssem, acc):
    _gs_body(table, src2, dst2, zeros2, agg, srca, dsta, stg2, oidx, rows,
             gsem, ssem, acc)


PK = 2504
PV = 2500


def _dinv_full(degb):
    deg = degb[0:PV] + degb[PV:2 * PV]
    dinv = jnp.where(deg > 0, lax.rsqrt(deg), 0.0)
    return jnp.concatenate([dinv, jnp.zeros((PK - PV, 128), F32)], axis=0)


def _pick(y, off):
    return jnp.concatenate(
        [y[:, 96 * k + off:96 * k + off + F] for k in range(4)], axis=1)


def _row_pad(a):
    return jnp.concatenate([a, jnp.zeros((PK - PV, 128), F32)], axis=0)


def _mm1_body(xp_ref, w_ref, db_ref, z_ref, t_ref, g_ref):
    y = jnp.dot(xp_ref[...], w_ref[...], preferred_element_type=F32,
                precision=lax.Precision.HIGHEST)
    z_ref[...] = _row_pad(_pick(y, 0))
    t_ref[...] = _row_pad(_pick(y, F))
    g_ref[...] = _row_pad(_dinv_full(db_ref[...])[0:PV] * _pick(y, 2 * F))


def _mid_body(t_ref, a_ref, db_ref, o_ref):
    dinv = _dinv_full(db_ref[...])
    o_ref[...] = dinv * t_ref[...] - 2.0 * dinv * dinv * a_ref[...]


def _row_mask():
    return lax.broadcasted_iota(jnp.int32, (PK, 128), 0) < PV


def _post_body(z_ref, a_ref, db_ref, b_ref, r_ref, p_ref, q_ref):
    y = z_ref[...] - _dinv_full(db_ref[...]) * a_ref[...] + b_ref[...]
    r = jnp.maximum(y, 0.0)
    r_ref[...] = r
    rm = jnp.where(_row_mask(), r, 0.0)
    p_ref[...] = jnp.broadcast_to(jnp.sum(rm, axis=0, keepdims=True),
                                  (8, 128))
    q_ref[...] = jnp.broadcast_to(jnp.sum(rm * rm, axis=0, keepdims=True),
                                  (8, 128))


def _fold4(s):
    return (s[:, 0:32] + s[:, 32:64] + s[:, 64:96] + s[:, 96:128])


def _rep4(s):
    return jnp.concatenate([s, s, s, s], axis=1)


def _stats(p_ref, q_ref):
    mu = _rep4(_fold4(jnp.sum(p_ref[...], axis=0, keepdims=True))) * (
        1.0 / (8 * N))
    ms = _rep4(_fold4(jnp.sum(q_ref[...], axis=0, keepdims=True))) * (
        1.0 / (8 * N))
    var = ms - mu * mu
    return mu, lax.rsqrt(var + EPS)


def _mm2_body(r_ref, p_ref, q_ref, w_ref, db_ref, z_ref, t_ref, g_ref):
    mu, sinv = _stats(p_ref, q_ref)
    h = (r_ref[...] - mu) * sinv
    y = jnp.dot(h, w_ref[...], preferred_element_type=F32,
                precision=lax.Precision.HIGHEST)
    z_ref[...] = _pick(y, 0)
    t_ref[...] = _pick(y, F)
    g_ref[...] = _dinv_full(db_ref[...]) * _pick(y, 2 * F)


def _norm_body(r_ref, p_ref, q_ref, o_ref):
    mu, sinv = _stats(p_ref, q_ref)
    o_ref[...] = (r_ref[...] - mu) * sinv


def _full(shape):
    return pl.BlockSpec(shape, lambda: tuple(0 for _ in shape))


def _pk_out(n_arrs):
    return [jax.ShapeDtypeStruct((PK, 128), F32) for _ in range(n_arrs)]


_PKS = _full((PK, 128))
_DBS = _full((2 * PV, 128))
_PQS = _full((8, 128))

_mm1_call = pl.pallas_call(
    _mm1_body,
    in_specs=[_full((PV, 4 * CIN)), _full((4 * CIN, 4 * 3 * F)), _DBS],
    out_specs=[_PKS] * 3,
    out_shape=_pk_out(3),
)

_mid_call = pl.pallas_call(
    _mid_body,
    in_specs=[_PKS, _PKS, _DBS],
    out_specs=[_PKS],
    out_shape=_pk_out(1),
)

_post_call = pl.pallas_call(
    _post_body,
    in_specs=[_PKS, _PKS, _DBS, _full((1, 128))],
    out_specs=[_PKS, _PQS, _PQS],
    out_shape=_pk_out(1) + [jax.ShapeDtypeStruct((8, 128), F32)] * 2,
)

_mm2_call = pl.pallas_call(
    _mm2_body,
    in_specs=[_PKS, _PQS, _PQS, _full((CIN, 4 * 3 * F)), _DBS],
    out_specs=[_PKS] * 3,
    out_shape=_pk_out(3),
)

_norm_call = pl.pallas_call(
    _norm_body,
    in_specs=[_PKS, _PQS, _PQS],
    out_specs=[_PKS],
    out_shape=_pk_out(1),
)


def kernel(x, edge_index, W1_0, W1_1, W1_2, b1, W2_0, W2_1, W2_2, b2):
    src2 = edge_index[0].reshape(ER, 128)
    dst2 = edge_index[1].reshape(ER, 128)
    zeros1 = jnp.zeros((PSUB,), F32)
    ones_h = jnp.ones((128,), F32)
    zeros2 = jnp.zeros((1024, HALF), F32)

    degb = _deg_kernel(src2, zeros1, ones_h)

    src5 = edge_index[0].reshape(625, 512)
    dst5 = edge_index[1].reshape(625, 512)

    def gs(gin):
        agg = _gs_kernel(gin.reshape(APAD, HALF), src5, dst5, zeros2)
        return agg.reshape(PK, 128)

    wcat1 = jnp.concatenate([W1_0 - W1_2, W1_1, W1_2], axis=1)
    w14 = jax.scipy.linalg.block_diag(wcat1, wcat1, wcat1, wcat1)
    xp = x.reshape(PV, 4 * CIN)
    b1p = jnp.tile(b1, 4).reshape(1, 128)
    b2p = jnp.tile(b2, 4).reshape(1, 128)
    wcat2 = jnp.concatenate([W2_0 - W2_2, W2_1, W2_2], axis=1)
    w4 = jax.scipy.linalg.block_diag(wcat2, wcat2, wcat2, wcat2)

    z0, t1, gin1 = _mm1_call(xp, w14, degb)
    agg1 = gs(gin1)
    (gin2,) = _mid_call(t1, agg1, degb)
    agg2 = gs(gin2)
    r1, p1, q1 = _post_call(z0, agg2, degb, b1p)

    z0b, t2, gin3 = _mm2_call(r1, p1, q1, w4, degb)
    agg3 = gs(gin3)
    (gin4,) = _mid_call(t2, agg3, degb)
    agg4 = gs(gin4)
    r2, p2, q2 = _post_call(z0b, agg4, degb, b2p)

    (outp,) = _norm_call(r2, p2, q2)
    return outp[:PV].reshape(N, F)
